# baseline (speedup 1.0000x reference)
import functools
import jax
import jax.numpy as jnp
from jax import lax
from jax.experimental import pallas as pl
from jax.experimental.pallas import tpu as pltpu
from jax.experimental.pallas import tpu_sc as plsc


def _chamfer_kernel(x1s_ref, x2s_ref, dist1_ref, dist2_ref):
    m_idx = pl.program_id(1)

    a = x1s_ref[0]
    b = x2s_ref[0]

    dims = (((0,), (0,)), ((), ()))
    d = jax.lax.dot_general(a, b, dims, preferred_element_type=jnp.float32)

    dist2_ref[0, 0] = jnp.min(d, axis=0)

    tm = d.shape[1]
    acc = d[:, 0:128]
    for j in range(1, tm // 128):
        acc = jnp.minimum(acc, d[:, j * 128:(j + 1) * 128])
    tile_min1 = jnp.min(acc.T, axis=0)

    @pl.when(m_idx == 0)
    def _init():
        dist1_ref[0, 0] = tile_min1

    @pl.when(m_idx != 0)
    def _acc():
        dist1_ref[0, 0] = jnp.minimum(dist1_ref[0, 0], tile_min1)


def _bf16_exact_split(x):
    hi = jax.lax.bitcast_convert_type(
        jax.lax.bitcast_convert_type(x, jnp.uint32) & jnp.uint32(0xFFFF0000),
        jnp.float32,
    )
    return hi, x - hi


def _tc_kernel(input1, input2):
    B, N, _ = input1.shape
    M = input2.shape[1]
    TM = 1024 if M % 1024 == 0 else 512

    x1t = jnp.transpose(input1, (0, 2, 1))
    x2t = jnp.transpose(input2, (0, 2, 1))

    sq1 = jnp.sum(input1 * input1, axis=-1)
    sq2 = jnp.sum(input2 * input2, axis=-1)
    sq1_hi, sq1_lo = _bf16_exact_split(sq1)
    sq2_hi, sq2_lo = _bf16_exact_split(sq2)

    ones1 = jnp.ones((B, 2, N), jnp.float32)
    zeros1 = jnp.zeros((B, 1, N), jnp.float32)
    x1s = jnp.concatenate(
        [-2.0 * x1t, sq1_hi[:, None, :], sq1_lo[:, None, :], ones1, zeros1],
        axis=1,
    )
    ones2 = jnp.ones((B, 2, M), jnp.float32)
    zeros2 = jnp.zeros((B, 1, M), jnp.float32)
    x2s = jnp.concatenate(
        [x2t, ones2, sq2_hi[:, None, :], sq2_lo[:, None, :], zeros2], axis=1
    )

    x1s = x1s.astype(jnp.bfloat16)
    x2s = x2s.astype(jnp.bfloat16)

    dist1, dist2 = pl.pallas_call(
        _chamfer_kernel,
        grid=(B, M // TM),
        in_specs=[
            pl.BlockSpec((1, 8, N), lambda b, m: (b, 0, 0)),
            pl.BlockSpec((1, 8, TM), lambda b, m: (b, 0, m)),
        ],
        out_specs=[
            pl.BlockSpec((1, 1, N), lambda b, m: (b, 0, 0)),
            pl.BlockSpec((1, 1, TM), lambda b, m: (b, 0, m)),
        ],
        out_shape=[
            jax.ShapeDtypeStruct((B, 1, N), jnp.float32),
            jax.ShapeDtypeStruct((B, 1, M), jnp.float32),
        ],
    )(x1s, x2s)

    return dist1[:, 0, :], dist2[:, 0, :]


B, N, MSC = 4, 4096, 512
NW = 32
RPW = (B * N) // NW // B * 1
WPB = NW // B
ROWS = N // WPB
L = 16


def _rnd(v):
    u = jax.lax.bitcast_convert_type(v, jnp.uint32)
    rounded = (u + jnp.uint32(0x7FFF) + ((u >> jnp.uint32(16)) & jnp.uint32(1))) & jnp.uint32(0xFFFF0000)
    return jax.lax.bitcast_convert_type(rounded, jnp.float32)


def make_sc_chamfer():
    mesh = plsc.VectorSubcoreMesh(core_axis_name="c", subcore_axis_name="s")

    @functools.partial(
        pl.kernel,
        mesh=mesh,
        compiler_params=pltpu.CompilerParams(needs_layout_passes=False),
        out_type=[
            jax.ShapeDtypeStruct((B, N), jnp.float32),
            jax.ShapeDtypeStruct((B, MSC), jnp.float32),
        ],
        scratch_types=[
            pltpu.VMEM((ROWS,), jnp.float32),
            pltpu.VMEM((ROWS,), jnp.float32),
            pltpu.VMEM((ROWS,), jnp.float32),
            pltpu.VMEM((MSC,), jnp.float32),
            pltpu.VMEM((MSC,), jnp.float32),
            pltpu.VMEM((MSC,), jnp.float32),
            pltpu.VMEM((MSC,), jnp.float32),
            pltpu.VMEM((ROWS,), jnp.float32),
            pltpu.VMEM((WPB, MSC), jnp.float32),
            pltpu.VMEM_SHARED((NW, MSC), jnp.float32),
        ],
    )
    def sc_chamfer(
        x1x_h, x1y_h, x1z_h, x2x_h, x2y_h, x2z_h,
        d1_h, d2_h,
        r1x, r1y, r1z, c2x, c2y, c2z, cmin, d1v, comb, shared,
    ):
        cid = lax.axis_index("c")
        sid = lax.axis_index("s")
        wid = cid * 16 + sid
        b = wid // WPB
        r0 = (wid % WPB) * ROWS

        pltpu.sync_copy(x1x_h.at[b, pl.ds(r0, ROWS)], r1x)
        pltpu.sync_copy(x1y_h.at[b, pl.ds(r0, ROWS)], r1y)
        pltpu.sync_copy(x1z_h.at[b, pl.ds(r0, ROWS)], r1z)
        pltpu.sync_copy(x2x_h.at[b, pl.ds(0, MSC)], c2x)
        pltpu.sync_copy(x2y_h.at[b, pl.ds(0, MSC)], c2y)
        pltpu.sync_copy(x2z_h.at[b, pl.ds(0, MSC)], c2z)

        inf = jnp.full((L,), jnp.inf, jnp.float32)
        for cv in range(MSC // L):
            cmin[pl.ds(cv * L, L)] = inf

        def row_block(cb, _):
            off = pl.multiple_of(cb * L, 8)
            rbx = r1x[pl.ds(off, L)]
            rby = r1y[pl.ds(off, L)]
            rbz = r1z[pl.ds(off, L)]
            for lane in range(L):
                li = jnp.full((L,), lane, jnp.int32)
                ax = jnp.take(rbx, li)
                ay = jnp.take(rby, li)
                az = jnp.take(rbz, li)
                s1 = (ax * ax + ay * ay) + az * az
                axr, ayr, azr = _rnd(ax), _rnd(ay), _rnd(az)
                racc = inf
                for cv in range(MSC // L):
                    sl = pl.ds(cv * L, L)
                    bx, by, bz = c2x[sl], c2y[sl], c2z[sl]
                    s2 = (bx * bx + by * by) + bz * bz
                    cr = (axr * _rnd(bx) + ayr * _rnd(by)) + azr * _rnd(bz)
                    d = (s1 + s2) - 2.0 * cr
                    racc = jnp.minimum(racc, d)
                    cmin[sl] = jnp.minimum(cmin[sl], d)
                lanes = lax.iota(jnp.int32, L)
                for sh in (8, 4, 2, 1):
                    racc = jnp.minimum(racc, jnp.take(racc, (lanes + sh) % L))
                lane0 = lanes == lane
                plsc.store_scatter(d1v, [jnp.full((L,), cb * L + lane, jnp.int32)],
                                   racc, mask=lane0)
            return _

        lax.fori_loop(0, ROWS // L, row_block, 0)

        pltpu.sync_copy(d1v, d1_h.at[b, pl.ds(r0, ROWS)])
        pltpu.sync_copy(cmin, shared.at[wid])
        plsc.subcore_barrier()

        @pl.when(wid % WPB == 0)
        def _combine():
            base = pl.multiple_of(b * WPB, 8)
            pltpu.sync_copy(shared.at[pl.ds(base, WPB)], comb)
            for cv in range(MSC // L):
                sl = pl.ds(cv * L, L)
                acc = comb[0, sl]
                for j in range(1, WPB):
                    acc = jnp.minimum(acc, comb[j, sl])
                cmin[sl] = acc
            pltpu.sync_copy(cmin, d2_h.at[b])

    return sc_chamfer


def sc_chamfer_slice(input1, input2_slice):
    f = make_sc_chamfer()
    x1 = jnp.transpose(input1, (0, 2, 1))
    x2 = jnp.transpose(input2_slice, (0, 2, 1))
    return f(x1[:, 0], x1[:, 1], x1[:, 2], x2[:, 0], x2[:, 1], x2[:, 2])


def kernel(input1, input2):
    M = input2.shape[1]
    i2_tc = input2[:, : M - MSC, :]
    i2_sc = input2[:, M - MSC :, :]
    d1_tc, d2_tc = _tc_kernel(input1, i2_tc)
    d1_sc, d2_sc = sc_chamfer_slice(input1, i2_sc)
    dist1 = jnp.minimum(d1_tc, d1_sc)
    dist2 = jnp.concatenate([d2_tc, d2_sc], axis=1)
    return dist1, dist2


# speedup vs baseline: 7.6853x; 7.6853x over previous
import jax
import jax.numpy as jnp
from jax.experimental import pallas as pl


def _chamfer_kernel(x1s_ref, x2s_ref, dist1_ref, dist2_ref):
    m_idx = pl.program_id(1)

    a = x1s_ref[0]
    b = x2s_ref[0]

    dims = (((0,), (0,)), ((), ()))
    d = jax.lax.dot_general(a, b, dims, preferred_element_type=jnp.float32)

    dist2_ref[0, 0] = jnp.min(d, axis=0)

    tm = d.shape[1]
    acc = d[:, 0:128]
    for j in range(1, tm // 128):
        acc = jnp.minimum(acc, d[:, j * 128:(j + 1) * 128])
    tile_min1 = jnp.min(acc.T, axis=0)

    @pl.when(m_idx == 0)
    def _init():
        dist1_ref[0, 0] = tile_min1

    @pl.when(m_idx != 0)
    def _acc():
        dist1_ref[0, 0] = jnp.minimum(dist1_ref[0, 0], tile_min1)


def _bf16_exact_split(x):
    hi = jax.lax.bitcast_convert_type(
        jax.lax.bitcast_convert_type(x, jnp.uint32) & jnp.uint32(0xFFFF0000),
        jnp.float32,
    )
    return hi, x - hi


def kernel(input1, input2):
    B, N, _ = input1.shape
    M = input2.shape[1]
    TM = 2048

    x1t = jnp.transpose(input1, (0, 2, 1))
    x2t = jnp.transpose(input2, (0, 2, 1))

    sq1 = jnp.sum(input1 * input1, axis=-1)
    sq2 = jnp.sum(input2 * input2, axis=-1)
    sq1_hi, sq1_lo = _bf16_exact_split(sq1)
    sq2_hi, sq2_lo = _bf16_exact_split(sq2)

    ones1 = jnp.ones((B, 2, N), jnp.float32)
    zeros1 = jnp.zeros((B, 1, N), jnp.float32)
    x1s = jnp.concatenate(
        [-2.0 * x1t, sq1_hi[:, None, :], sq1_lo[:, None, :], ones1, zeros1],
        axis=1,
    )
    ones2 = jnp.ones((B, 2, M), jnp.float32)
    zeros2 = jnp.zeros((B, 1, M), jnp.float32)
    x2s = jnp.concatenate(
        [x2t, ones2, sq2_hi[:, None, :], sq2_lo[:, None, :], zeros2], axis=1
    )

    x1s = x1s.astype(jnp.bfloat16)
    x2s = x2s.astype(jnp.bfloat16)

    dist1, dist2 = pl.pallas_call(
        _chamfer_kernel,
        grid=(B, M // TM),
        in_specs=[
            pl.BlockSpec((1, 8, N), lambda b, m: (b, 0, 0)),
            pl.BlockSpec((1, 8, TM), lambda b, m: (b, 0, m)),
        ],
        out_specs=[
            pl.BlockSpec((1, 1, N), lambda b, m: (b, 0, 0)),
            pl.BlockSpec((1, 1, TM), lambda b, m: (b, 0, m)),
        ],
        out_shape=[
            jax.ShapeDtypeStruct((B, 1, N), jnp.float32),
            jax.ShapeDtypeStruct((B, 1, M), jnp.float32),
        ],
    )(x1s, x2s)

    return dist1[:, 0, :], dist2[:, 0, :]
